# baseline (speedup 1.0000x reference)
import functools

import jax
import jax.numpy as jnp
from jax import lax
from jax.experimental import pallas as pl
from jax.experimental.pallas import tpu as pltpu
from jax.experimental.pallas import tpu_sc as plsc

VOCAB = 1000000
DIM = 64
B = 4096
L = 50

N = B * L
NC, NS = 2, 16
NW = NC * NS
PER_W = N // NW
CHUNK = 800
NCHUNK = PER_W // CHUNK


def _gather_kernel(emb_hbm, idx_hbm, out_hbm,
                   idx_v0, idx_v1, rows_v0, rows_v1, sem0, sem1):
    wid = lax.axis_index("s") * NC + lax.axis_index("c")
    base = wid * PER_W
    idx_bufs = (idx_v0, idx_v1)
    rows_bufs = (rows_v0, rows_v1)
    sems = (sem0, sem1)

    pltpu.sync_copy(idx_hbm.at[pl.ds(base, CHUNK)], idx_v0)
    pltpu.async_copy(emb_hbm.at[idx_v0], rows_v0, sem0)

    for j in range(NCHUNK):
        cur = j % 2
        nxt = (j + 1) % 2
        if j + 1 < NCHUNK:
            pltpu.sync_copy(
                idx_hbm.at[pl.ds(base + (j + 1) * CHUNK, CHUNK)],
                idx_bufs[nxt])
            pltpu.async_copy(emb_hbm.at[idx_bufs[nxt]], rows_bufs[nxt],
                             sems[nxt])
        pltpu.make_async_copy(emb_hbm.at[idx_bufs[cur]], rows_bufs[cur],
                              sems[cur]).wait()
        pltpu.sync_copy(rows_bufs[cur],
                        out_hbm.at[pl.ds(base + j * CHUNK, CHUNK)])


@jax.jit
def _embed_lookup(emb, idx_flat):
    mesh = plsc.VectorSubcoreMesh(core_axis_name="c", subcore_axis_name="s")
    k = pl.kernel(
        _gather_kernel,
        mesh=mesh,
        compiler_params=pltpu.CompilerParams(use_tc_tiling_on_sc=False),
        out_type=jax.ShapeDtypeStruct((N, DIM), jnp.float32),
        scratch_types=[
            pltpu.VMEM((CHUNK,), jnp.int32),
            pltpu.VMEM((CHUNK,), jnp.int32),
            pltpu.VMEM((CHUNK, DIM), jnp.float32),
            pltpu.VMEM((CHUNK, DIM), jnp.float32),
            pltpu.SemaphoreType.DMA,
            pltpu.SemaphoreType.DMA,
        ],
    )
    return k(emb, idx_flat)


def kernel(emb, inputs):
    out = _embed_lookup(emb, inputs.reshape(N))
    return out.reshape(B, L, DIM)
